# baseline (device time: 18258 ns/iter reference)
import jax
import jax.numpy as jnp
from jax import lax
from jax.experimental import pallas as pl
from jax.experimental.pallas import tpu as pltpu

N_DEV = 4


def kernel(partial, resid, gamma):
    x = partial[0]
    g = gamma.reshape(1, -1)
    m, n = x.shape
    half = m // 2
    q = m // 4
    e = m // 8

    def body(x_ref, resid_ref, g_ref, out_ref,
             xb_ref, r1a, r1b, s2a, r2a, s2b, r2b, norm_bf,
             send_sems, recv_sems):
        my = lax.axis_index("i")
        pY = my ^ 1
        pX = 3 - my
        my_x = my // 2
        my_y = (my ^ (my >> 1)) & 1

        def exchange(src, dst, sem_idx, dev):
            rdma = pltpu.make_async_remote_copy(
                src_ref=src, dst_ref=dst,
                send_sem=send_sems.at[sem_idx],
                recv_sem=recv_sems.at[sem_idx],
                device_id=(dev,),
                device_id_type=pl.DeviceIdType.MESH,
            )
            rdma.start()
            return rdma

        barrier = pltpu.get_barrier_semaphore()
        for p in (pY, pX):
            pl.semaphore_signal(barrier, inc=1, device_id=(p,),
                                device_id_type=pl.DeviceIdType.MESH)
        xb_ref[:, :] = x_ref[:, :].astype(jnp.bfloat16)
        pl.semaphore_wait(barrier, 2)

        keepA1 = my_y * q
        compA1 = (1 - my_y) * q
        keepB1 = half + my_x * q
        compB1 = half + (1 - my_x) * q
        rowA = keepA1 + my_x * e
        rowB = keepB1 + my_y * e

        r1 = [
            exchange(xb_ref.at[pl.ds(compA1, q), :], r1a, (0, 0), pY),
            exchange(xb_ref.at[pl.ds(compB1, q), :], r1b, (0, 1), pX),
        ]

        r1[0].wait_recv()
        s2a[:, :] = (
            x_ref[pl.ds(keepA1 + (1 - my_x) * e, e), :]
            + r1a[pl.ds((1 - my_x) * e, e), :].astype(jnp.float32)
        ).astype(jnp.bfloat16)
        r2_a = exchange(s2a, r2a, (1, 0), pX)

        r1[1].wait_recv()
        s2b[:, :] = (
            x_ref[pl.ds(keepB1 + (1 - my_y) * e, e), :]
            + r1b[pl.ds((1 - my_y) * e, e), :].astype(jnp.float32)
        ).astype(jnp.bfloat16)
        r2_b = exchange(s2b, r2b, (1, 1), pY)

        sA_keep = (x_ref[pl.ds(rowA, e), :]
                   + r1a[pl.ds(my_x * e, e), :].astype(jnp.float32))
        sB_keep = (x_ref[pl.ds(rowB, e), :]
                   + r1b[pl.ds(my_y * e, e), :].astype(jnp.float32))

        def norm_store(y, rows):
            rms = jnp.sqrt(jnp.mean(y * y, axis=-1, keepdims=True) + 1e-6)
            norm_bf[pl.ds(rows, e), :] = (
                y / rms * g_ref[0, :]).astype(jnp.bfloat16)

        r2_a.wait_recv()
        norm_store(sA_keep + r2a[:, :].astype(jnp.float32)
                   + resid_ref[pl.ds(rowA, e), :], rowA)
        r3_a = exchange(norm_bf.at[pl.ds(rowA, e), :],
                        norm_bf.at[pl.ds(rowA, e), :], (2, 0), pX)

        r2_b.wait_recv()
        norm_store(sB_keep + r2b[:, :].astype(jnp.float32)
                   + resid_ref[pl.ds(rowB, e), :], rowB)
        r3_b = exchange(norm_bf.at[pl.ds(rowB, e), :],
                        norm_bf.at[pl.ds(rowB, e), :], (2, 1), pY)

        recv3a = pltpu.make_async_remote_copy(
            src_ref=norm_bf.at[pl.ds(keepA1 + (1 - my_x) * e, e), :],
            dst_ref=norm_bf.at[pl.ds(keepA1 + (1 - my_x) * e, e), :],
            send_sem=send_sems.at[2, 0], recv_sem=recv_sems.at[2, 0],
            device_id=(pX,), device_id_type=pl.DeviceIdType.MESH,
        )
        recv3a.wait_recv()
        r4_a = exchange(norm_bf.at[pl.ds(keepA1, q), :],
                        norm_bf.at[pl.ds(keepA1, q), :], (3, 0), pY)

        recv3b = pltpu.make_async_remote_copy(
            src_ref=norm_bf.at[pl.ds(keepB1 + (1 - my_y) * e, e), :],
            dst_ref=norm_bf.at[pl.ds(keepB1 + (1 - my_y) * e, e), :],
            send_sem=send_sems.at[2, 1], recv_sem=recv_sems.at[2, 1],
            device_id=(pY,), device_id_type=pl.DeviceIdType.MESH,
        )
        recv3b.wait_recv()
        r4_b = exchange(norm_bf.at[pl.ds(keepB1, q), :],
                        norm_bf.at[pl.ds(keepB1, q), :], (3, 1), pX)

        recv4a = pltpu.make_async_remote_copy(
            src_ref=norm_bf.at[pl.ds(compA1, q), :],
            dst_ref=norm_bf.at[pl.ds(compA1, q), :],
            send_sem=send_sems.at[3, 0], recv_sem=recv_sems.at[3, 0],
            device_id=(pY,), device_id_type=pl.DeviceIdType.MESH,
        )
        recv4a.wait_recv()
        recv4b = pltpu.make_async_remote_copy(
            src_ref=norm_bf.at[pl.ds(compB1, q), :],
            dst_ref=norm_bf.at[pl.ds(compB1, q), :],
            send_sem=send_sems.at[3, 1], recv_sem=recv_sems.at[3, 1],
            device_id=(pX,), device_id_type=pl.DeviceIdType.MESH,
        )
        recv4b.wait_recv()

        out_ref[:, :] = norm_bf[:, :].astype(jnp.float32)

        for rdma in r1 + [r2_a, r2_b, r3_a, r3_b, r4_a, r4_b]:
            rdma.wait_send()

    return pl.pallas_call(
        body,
        out_shape=jax.ShapeDtypeStruct((m, n), jnp.float32),
        in_specs=[
            pl.BlockSpec(memory_space=pltpu.VMEM),
            pl.BlockSpec(memory_space=pltpu.VMEM),
            pl.BlockSpec(memory_space=pltpu.VMEM),
        ],
        out_specs=pl.BlockSpec(memory_space=pltpu.VMEM),
        scratch_shapes=[
            pltpu.VMEM((m, n), jnp.bfloat16),
            pltpu.VMEM((q, n), jnp.bfloat16),
            pltpu.VMEM((q, n), jnp.bfloat16),
            pltpu.VMEM((e, n), jnp.bfloat16),
            pltpu.VMEM((e, n), jnp.bfloat16),
            pltpu.VMEM((e, n), jnp.bfloat16),
            pltpu.VMEM((e, n), jnp.bfloat16),
            pltpu.VMEM((m, n), jnp.bfloat16),
            pltpu.SemaphoreType.DMA((4, 2)),
            pltpu.SemaphoreType.DMA((4, 2)),
        ],
        compiler_params=pltpu.CompilerParams(collective_id=0),
    )(x, resid, g)


# device time: 15604 ns/iter; 1.1701x vs baseline; 1.1701x over previous
import jax
import jax.numpy as jnp
from jax import lax
from jax.experimental import pallas as pl
from jax.experimental.pallas import tpu as pltpu

N_DEV = 4
NSUB = 4


def kernel(partial, resid, gamma):
    x = partial[0]
    g = gamma.reshape(1, -1)
    m, n = x.shape
    blk = m // N_DEV
    sub = blk // NSUB

    def body(x_ref, resid_ref, g_ref, out_ref,
             xb_ref, rs_buf, ag_buf,
             send_sems1, recv_sems1, send_sems2, recv_sems2):
        my = lax.axis_index("i")

        barrier = pltpu.get_barrier_semaphore()
        for k in range(1, N_DEV):
            p = (my + k) % N_DEV
            pl.semaphore_signal(barrier, inc=1, device_id=(p,),
                                device_id_type=pl.DeviceIdType.MESH)

        xb_ref[:, :] = x_ref[:, :].astype(jnp.bfloat16)
        for s in range(NSUB):
            rs_buf[pl.ds(my, 1), s] = jnp.zeros((1, sub, n), jnp.bfloat16)

        pl.semaphore_wait(barrier, N_DEV - 1)

        rs_sends = []
        for s in range(NSUB):
            for k in range(1, N_DEV):
                p = (my + k) % N_DEV
                rdma = pltpu.make_async_remote_copy(
                    src_ref=xb_ref.at[pl.ds(p * blk + s * sub, sub), :],
                    dst_ref=rs_buf.at[my, s],
                    send_sem=send_sems1.at[k - 1, s],
                    recv_sem=recv_sems1.at[my, s],
                    device_id=(p,),
                    device_id_type=pl.DeviceIdType.MESH,
                )
                rdma.start()
                rs_sends.append(rdma)

        ag_sends = []
        for s in range(NSUB):
            for k in range(1, N_DEV):
                q = (my + k) % N_DEV
                recv = pltpu.make_async_remote_copy(
                    src_ref=rs_buf.at[q, s],
                    dst_ref=rs_buf.at[q, s],
                    send_sem=send_sems1.at[k - 1, s],
                    recv_sem=recv_sems1.at[q, s],
                    device_id=(q,),
                    device_id_type=pl.DeviceIdType.MESH,
                )
                recv.wait_recv()

            rows = pl.ds(my * blk + s * sub, sub)
            y = (x_ref[rows, :]
                 + (rs_buf[0, s] + rs_buf[1, s]
                    + rs_buf[2, s] + rs_buf[3, s]).astype(jnp.float32)
                 + resid_ref[rows, :])
            rms = jnp.sqrt(jnp.mean(y * y, axis=-1, keepdims=True) + 1e-6)
            z = y / rms * g_ref[0, :]
            out_ref[rows, :] = z
            ag_buf[pl.ds(my, 1), s] = z.astype(jnp.bfloat16)[None]

            for k in range(1, N_DEV):
                p = (my + k) % N_DEV
                rdma = pltpu.make_async_remote_copy(
                    src_ref=ag_buf.at[my, s],
                    dst_ref=ag_buf.at[my, s],
                    send_sem=send_sems2.at[k - 1, s],
                    recv_sem=recv_sems2.at[my, s],
                    device_id=(p,),
                    device_id_type=pl.DeviceIdType.MESH,
                )
                rdma.start()
                ag_sends.append(rdma)

        for s in range(NSUB):
            for k in range(1, N_DEV):
                q = (my + k) % N_DEV
                recv = pltpu.make_async_remote_copy(
                    src_ref=ag_buf.at[q, s],
                    dst_ref=ag_buf.at[q, s],
                    send_sem=send_sems2.at[k - 1, s],
                    recv_sem=recv_sems2.at[q, s],
                    device_id=(q,),
                    device_id_type=pl.DeviceIdType.MESH,
                )
                recv.wait_recv()
                out_ref[pl.ds(q * blk + s * sub, sub), :] = (
                    ag_buf[pl.ds(q, 1), s].astype(jnp.float32)[0])

        for rdma in rs_sends + ag_sends:
            rdma.wait_send()

    return pl.pallas_call(
        body,
        out_shape=jax.ShapeDtypeStruct((m, n), jnp.float32),
        in_specs=[
            pl.BlockSpec(memory_space=pltpu.VMEM),
            pl.BlockSpec(memory_space=pltpu.VMEM),
            pl.BlockSpec(memory_space=pltpu.VMEM),
        ],
        out_specs=pl.BlockSpec(memory_space=pltpu.VMEM),
        scratch_shapes=[
            pltpu.VMEM((m, n), jnp.bfloat16),
            pltpu.VMEM((N_DEV, NSUB, sub, n), jnp.bfloat16),
            pltpu.VMEM((N_DEV, NSUB, sub, n), jnp.bfloat16),
            pltpu.SemaphoreType.DMA((N_DEV - 1, NSUB)),
            pltpu.SemaphoreType.DMA((N_DEV, NSUB)),
            pltpu.SemaphoreType.DMA((N_DEV - 1, NSUB)),
            pltpu.SemaphoreType.DMA((N_DEV, NSUB)),
        ],
        compiler_params=pltpu.CompilerParams(collective_id=0),
    )(x, resid, g)


# device time: 15523 ns/iter; 1.1762x vs baseline; 1.0052x over previous
import jax
import jax.numpy as jnp
from jax import lax
from jax.experimental import pallas as pl
from jax.experimental.pallas import tpu as pltpu

N_DEV = 4
NSUB = 4


def kernel(partial, resid, gamma):
    x = partial[0]
    g = gamma.reshape(1, -1)
    m, n = x.shape
    blk = m // N_DEV
    sub = blk // NSUB

    def body(x_ref, resid_ref, g_ref, out_ref,
             xb_ref, rs_buf, ag_buf,
             send_sems1, recv_sems1, send_sems2, recv_sems2):
        my = lax.axis_index("i")

        barrier = pltpu.get_barrier_semaphore()
        for k in range(1, N_DEV):
            p = (my + k) % N_DEV
            pl.semaphore_signal(barrier, inc=1, device_id=(p,),
                                device_id_type=pl.DeviceIdType.MESH)

        xb_ref[:, :] = x_ref[:, :].astype(jnp.bfloat16)
        for s in range(NSUB):
            rs_buf[pl.ds(my, 1), s] = jnp.zeros((1, sub, n), jnp.bfloat16)

        pl.semaphore_wait(barrier, N_DEV - 1)

        rs_sends = []
        for s in range(NSUB):
            for k in (2, 1, 3):
                p = (my + k) % N_DEV
                rdma = pltpu.make_async_remote_copy(
                    src_ref=xb_ref.at[pl.ds(p * blk + s * sub, sub), :],
                    dst_ref=rs_buf.at[my, s],
                    send_sem=send_sems1.at[k - 1, s],
                    recv_sem=recv_sems1.at[my, s],
                    device_id=(p,),
                    device_id_type=pl.DeviceIdType.MESH,
                )
                rdma.start()
                rs_sends.append(rdma)

        ag_sends = []
        for s in range(NSUB):
            for k in range(1, N_DEV):
                q = (my + k) % N_DEV
                recv = pltpu.make_async_remote_copy(
                    src_ref=rs_buf.at[q, s],
                    dst_ref=rs_buf.at[q, s],
                    send_sem=send_sems1.at[k - 1, s],
                    recv_sem=recv_sems1.at[q, s],
                    device_id=(q,),
                    device_id_type=pl.DeviceIdType.MESH,
                )
                recv.wait_recv()

            rows = pl.ds(my * blk + s * sub, sub)
            y = (x_ref[rows, :]
                 + (rs_buf[0, s] + rs_buf[1, s]
                    + rs_buf[2, s] + rs_buf[3, s]).astype(jnp.float32)
                 + resid_ref[rows, :])
            rms = jnp.sqrt(jnp.mean(y * y, axis=-1, keepdims=True) + 1e-6)
            z = y / rms * g_ref[0, :]
            ag_buf[pl.ds(my, 1), s] = z.astype(jnp.bfloat16)[None]

            for k in (2, 1, 3):
                p = (my + k) % N_DEV
                rdma = pltpu.make_async_remote_copy(
                    src_ref=ag_buf.at[my, s],
                    dst_ref=ag_buf.at[my, s],
                    send_sem=send_sems2.at[k - 1, s],
                    recv_sem=recv_sems2.at[my, s],
                    device_id=(p,),
                    device_id_type=pl.DeviceIdType.MESH,
                )
                rdma.start()
                ag_sends.append(rdma)

            out_ref[rows, :] = z

        for s in range(NSUB):
            for k in range(1, N_DEV):
                q = (my + k) % N_DEV
                recv = pltpu.make_async_remote_copy(
                    src_ref=ag_buf.at[q, s],
                    dst_ref=ag_buf.at[q, s],
                    send_sem=send_sems2.at[k - 1, s],
                    recv_sem=recv_sems2.at[q, s],
                    device_id=(q,),
                    device_id_type=pl.DeviceIdType.MESH,
                )
                recv.wait_recv()
                out_ref[pl.ds(q * blk + s * sub, sub), :] = (
                    ag_buf[pl.ds(q, 1), s].astype(jnp.float32)[0])

        for rdma in rs_sends + ag_sends:
            rdma.wait_send()

    return pl.pallas_call(
        body,
        out_shape=jax.ShapeDtypeStruct((m, n), jnp.float32),
        in_specs=[
            pl.BlockSpec(memory_space=pltpu.VMEM),
            pl.BlockSpec(memory_space=pltpu.VMEM),
            pl.BlockSpec(memory_space=pltpu.VMEM),
        ],
        out_specs=pl.BlockSpec(memory_space=pltpu.VMEM),
        scratch_shapes=[
            pltpu.VMEM((m, n), jnp.bfloat16),
            pltpu.VMEM((N_DEV, NSUB, sub, n), jnp.bfloat16),
            pltpu.VMEM((N_DEV, NSUB, sub, n), jnp.bfloat16),
            pltpu.SemaphoreType.DMA((N_DEV - 1, NSUB)),
            pltpu.SemaphoreType.DMA((N_DEV, NSUB)),
            pltpu.SemaphoreType.DMA((N_DEV - 1, NSUB)),
            pltpu.SemaphoreType.DMA((N_DEV, NSUB)),
        ],
        compiler_params=pltpu.CompilerParams(collective_id=0),
    )(x, resid, g)
